# baseline (speedup 1.0000x reference)
import dataclasses
import functools

import jax
import jax.numpy as jnp
from jax import lax
from jax.experimental import pallas as pl
from jax.experimental.pallas import tpu as pltpu
from jax.experimental.pallas import tpu_sc as plsc

N = 10000
E = 320000
FD = 128

NC = 2
NS = 16
NW = NC * NS
EPW = E // NW
C = 80
NCH = EPW // C

NPAD = 10240
RPM = NPAD // NS

ZROWS = 128
RPT = NPAD // NS

_mesh = plsc.VectorSubcoreMesh(core_axis_name="c", subcore_axis_name="s")

_sc_params = pltpu.CompilerParams()
if "needs_layout_passes" in pltpu.CompilerParams.__dataclass_fields__:
    _sc_params = dataclasses.replace(_sc_params, needs_layout_passes=False)


@functools.partial(
    pl.kernel,
    out_type=jax.ShapeDtypeStruct((NC, NPAD), jnp.float32),
    mesh=_mesh,
    scratch_types=[
        pltpu.VMEM((EPW,), jnp.int32),
        pltpu.VMEM((EPW,), jnp.float32),
        pltpu.VMEM((NPAD,), jnp.float32),
        pltpu.VMEM((NS, RPM), jnp.float32),
        pltpu.VMEM_SHARED((NS, NPAD), jnp.float32),
        pltpu.SemaphoreType.DMA,
    ],
    compiler_params=_sc_params,
)
def _deg_kernel(col_hbm, w_hbm, out_hbm, col_v, w_v, hist_v, mrg_v, stage_sh,
                sem):
    cid = lax.axis_index("c")
    sid = lax.axis_index("s")
    wid = cid * NS + sid

    pltpu.async_copy(col_hbm.at[wid], col_v, sem)
    pltpu.async_copy(w_hbm.at[wid], w_v, sem)

    zero16 = jnp.zeros((16,), jnp.float32)

    @pl.loop(0, NPAD, step=16)
    def _(i):
        hist_v[pl.ds(i, 16)] = zero16

    pltpu.make_async_copy(col_hbm.at[wid], col_v, sem).wait()
    pltpu.make_async_copy(w_hbm.at[wid], w_v, sem).wait()

    @pl.loop(0, EPW, step=C)
    def _(e):
        for k in range(C // 16):
            idx = col_v[pl.ds(e + k * 16, 16)]
            val = w_v[pl.ds(e + k * 16, 16)]
            plsc.addupdate_scatter(hist_v, [idx], val)

    pltpu.sync_copy(hist_v, stage_sh.at[sid])
    plsc.subcore_barrier()
    for t in range(NS):
        pltpu.sync_copy(stage_sh.at[t, pl.ds(sid * RPM, RPM)], mrg_v.at[t])

    @pl.loop(0, RPM, step=16)
    def _(v):
        acc = mrg_v[0, pl.ds(v, 16)]
        for t in range(1, NS):
            acc = acc + mrg_v[t, pl.ds(v, 16)]
        mrg_v[0, pl.ds(v, 16)] = acc

    pltpu.sync_copy(mrg_v.at[0], out_hbm.at[cid, pl.ds(sid * RPM, RPM)])


@functools.partial(
    pl.kernel,
    out_type=jax.ShapeDtypeStruct((NC, NPAD, FD), jnp.float32),
    mesh=_mesh,
    scratch_types=[
        pltpu.VMEM((3, C), jnp.int32),
        pltpu.VMEM((3, C), jnp.int32),
        pltpu.VMEM((3, C), jnp.int32),
        pltpu.VMEM((3, C), jnp.int32),
        pltpu.VMEM((C, FD), jnp.float32),
        pltpu.VMEM((C, FD), jnp.float32),
        pltpu.VMEM((C, FD), jnp.float32),
        pltpu.VMEM((C, FD), jnp.float32),
        pltpu.VMEM_SHARED((NPAD, FD), jnp.float32),
        pltpu.SemaphoreType.DMA,
        pltpu.SemaphoreType.DMA,
        pltpu.SemaphoreType.DMA,
        pltpu.SemaphoreType.DMA,
        pltpu.SemaphoreType.DMA,
        pltpu.SemaphoreType.DMA,
    ],
    compiler_params=_sc_params,
)
def _agg_kernel(y_hbm, pk_hbm, out_hbm, pk0, pk1, pk2, pk3, gb0, gb1, gb2,
                gb3, acc_sh, gs0, gs1, ss0, ss1, ps0, ps1):
    cid = lax.axis_index("c")
    sid = lax.axis_index("s")
    wid = cid * NS + sid

    pkb = (pk0, pk1, pk2, pk3)
    gbb = (gb0, gb1, gb2, gb3)
    pss = (ps0, ps1)
    gss = (gs0, gs1)
    sss = (ss0, ss1)

    def issue_pk(j, q):
        pltpu.async_copy(pk_hbm.at[wid, j], pkb[q], pss[q % 2])

    def wait_pk(j, q):
        pltpu.make_async_copy(pk_hbm.at[wid, j], pkb[q], pss[q % 2]).wait()

    def issue_gather(q):
        pltpu.async_copy(y_hbm.at[pkb[q].at[0]], gbb[q], gss[q % 2])

    def wait_gather(q):
        pltpu.make_async_copy(y_hbm.at[pkb[q].at[0]], gbb[q],
                              gss[q % 2]).wait()

    def issue_scat(q):
        pltpu.async_copy(gbb[q], acc_sh.at[pkb[q].at[1]], sss[q % 2],
                         add=True)

    def wait_scat(q):
        pltpu.make_async_copy(gbb[q], acc_sh.at[pkb[q].at[1]],
                              sss[q % 2]).wait()

    issue_pk(0, 0)
    issue_pk(1, 1)

    zero16 = jnp.zeros((16,), jnp.float32)

    @pl.loop(0, C)
    def _(r):
        for k in range(FD // 16):
            gb0[r, pl.ds(k * 16, 16)] = zero16

    for z in range(RPT // C):
        r0 = sid * RPT + z * C
        pltpu.sync_copy(gb0, acc_sh.at[pl.ds(r0, C)])

    wait_pk(0, 0)
    issue_gather(0)
    plsc.subcore_barrier()

    def scale(q):
        gb = gbb[q]
        wrow = pkb[q]

        @pl.loop(0, C, step=16)
        def _(g):
            wv = plsc.bitcast(wrow[2, pl.ds(g, 16)], jnp.float32)
            for l in range(16):
                w_s = wv[l]
                for k in range(FD // 16):
                    gb[g + l, pl.ds(k * 16, 16)] = (
                        gb[g + l, pl.ds(k * 16, 16)] * w_s)

    def chunk(j, q, first, last):
        if not first:
            wait_scat((q + 2) % 4)
        wait_gather(q)
        scale(q)
        issue_scat(q)
        if not last:
            @pl.when(j + 2 < NCH)
            def _():
                issue_pk(j + 2, (q + 2) % 4)

            @pl.when(j + 1 < NCH)
            def _():
                wait_pk(j + 1, (q + 1) % 4)
                issue_gather((q + 1) % 4)

    chunk(0, 0, True, False)
    chunk(1, 1, True, False)

    @pl.loop(2, NCH - 3, step=4)
    def _(j):
        chunk(j, 2, False, False)
        chunk(j + 1, 3, False, False)
        chunk(j + 2, 0, False, False)
        chunk(j + 3, 1, False, False)

    chunk(NCH - 3, 2, False, False)
    chunk(NCH - 2, 3, False, False)
    chunk(NCH - 1, 0, False, True)
    wait_scat(3)
    wait_scat(0)

    plsc.subcore_barrier()
    for z in range(RPT // ZROWS):
        r0 = sid * RPT + z * ZROWS
        pltpu.sync_copy(acc_sh.at[pl.ds(r0, ZROWS)],
                        out_hbm.at[cid, pl.ds(r0, ZROWS)])


_RB = 2000
_GRID = (N // _RB,)


def _rows2(i):
    return (i, 0)


def _const2(i):
    return (0, 0)


_blk_feat = pl.BlockSpec((_RB, FD), _rows2)
_blk_w = pl.BlockSpec((FD, FD), _const2)
_blk_dis = pl.BlockSpec((_RB, 1), _rows2)
_blk_b = pl.BlockSpec((1, FD), _const2)


def _t1_body(x_ref, w_ref, d0_ref, d1_ref, y_ref, dis_ref):
    deg = d0_ref[...] + d1_ref[...]
    dis = jnp.where(deg > 0, lax.rsqrt(jnp.maximum(deg, 1e-12)), 0.0)
    dis_ref[...] = dis
    xw = jnp.dot(x_ref[...], w_ref[...], preferred_element_type=jnp.float32,
                 precision=lax.Precision.HIGHEST)
    y_ref[...] = xw * dis


_t1 = pl.pallas_call(
    _t1_body,
    grid=_GRID,
    in_specs=[_blk_feat, _blk_w, _blk_dis, _blk_dis],
    out_specs=[_blk_feat, _blk_dis],
    out_shape=[jax.ShapeDtypeStruct((N, FD), jnp.float32),
               jax.ShapeDtypeStruct((N, 1), jnp.float32)],
)


def _t2_body(p0_ref, p1_ref, dis_ref, b_ref, w_ref, y_ref):
    dis = dis_ref[...]
    h = dis * (p0_ref[...] + p1_ref[...]) + b_ref[...]
    h = jnp.maximum(h, 0.0)
    hw = jnp.dot(h, w_ref[...], preferred_element_type=jnp.float32,
                 precision=lax.Precision.HIGHEST)
    y_ref[...] = hw * dis


_t2 = pl.pallas_call(
    _t2_body,
    grid=_GRID,
    in_specs=[_blk_feat, _blk_feat, _blk_dis, _blk_b, _blk_w],
    out_specs=_blk_feat,
    out_shape=jax.ShapeDtypeStruct((N, FD), jnp.float32),
)


def _t4_body(p0_ref, p1_ref, dis_ref, b_ref, o_ref):
    o_ref[...] = dis_ref[...] * (p0_ref[...] + p1_ref[...]) + b_ref[...]


_t4 = pl.pallas_call(
    _t4_body,
    grid=_GRID,
    in_specs=[_blk_feat, _blk_feat, _blk_dis, _blk_b],
    out_specs=_blk_feat,
    out_shape=jax.ShapeDtypeStruct((N, FD), jnp.float32),
)


def kernel(x, edge_index, edge_weight, W1, b1, W2, b2, W3, b3):
    col2 = edge_index[1].reshape(NW, EPW)
    w2 = edge_weight.reshape(NW, EPW)
    pk = jnp.concatenate(
        [edge_index[0].reshape(NW, NCH, 1, C),
         edge_index[1].reshape(NW, NCH, 1, C),
         lax.bitcast_convert_type(edge_weight, jnp.int32).reshape(
             NW, NCH, 1, C)],
        axis=2)

    degp = _deg_kernel(col2, w2)
    d0 = degp[0, :N].reshape(N, 1)
    d1 = degp[1, :N].reshape(N, 1)

    y1, dis = _t1(x, W1, d0, d1)
    p1 = _agg_kernel(y1, pk)
    y2 = _t2(p1[0, :N], p1[1, :N], dis, b1.reshape(1, FD), W2)
    p2 = _agg_kernel(y2, pk)
    y3 = _t2(p2[0, :N], p2[1, :N], dis, b2.reshape(1, FD), W3)
    p3 = _agg_kernel(y3, pk)
    return _t4(p3[0, :N], p3[1, :N], dis, b3.reshape(1, FD))


# speedup vs baseline: 1.3728x; 1.3728x over previous
import dataclasses
import functools

import jax
import jax.numpy as jnp
from jax import lax
from jax.experimental import pallas as pl
from jax.experimental.pallas import tpu as pltpu
from jax.experimental.pallas import tpu_sc as plsc

N = 10000
E = 320000
FD = 128

NC = 2
NS = 16
NW = NC * NS
EPW = E // NW
C = 80
NCH = EPW // C

NPAD = 10240
RPM = NPAD // NS

ZROWS = 128
RPT = NPAD // NS

_mesh = plsc.VectorSubcoreMesh(core_axis_name="c", subcore_axis_name="s")

_sc_params = pltpu.CompilerParams()
if "needs_layout_passes" in pltpu.CompilerParams.__dataclass_fields__:
    _sc_params = dataclasses.replace(_sc_params, needs_layout_passes=False)


@functools.partial(
    pl.kernel,
    out_type=jax.ShapeDtypeStruct((NC, NPAD), jnp.float32),
    mesh=_mesh,
    scratch_types=[
        pltpu.VMEM((EPW,), jnp.int32),
        pltpu.VMEM((EPW,), jnp.float32),
        pltpu.VMEM((NPAD,), jnp.float32),
        pltpu.VMEM((NS, RPM), jnp.float32),
        pltpu.VMEM_SHARED((NS, NPAD), jnp.float32),
        pltpu.SemaphoreType.DMA,
    ],
    compiler_params=_sc_params,
)
def _deg_kernel(col_hbm, w_hbm, out_hbm, col_v, w_v, hist_v, mrg_v, stage_sh,
                sem):
    cid = lax.axis_index("c")
    sid = lax.axis_index("s")
    wid = cid * NS + sid

    pltpu.async_copy(col_hbm.at[wid], col_v, sem)
    pltpu.async_copy(w_hbm.at[wid], w_v, sem)

    zero16 = jnp.zeros((16,), jnp.float32)

    @pl.loop(0, NPAD, step=16)
    def _(i):
        hist_v[pl.ds(i, 16)] = zero16

    pltpu.make_async_copy(col_hbm.at[wid], col_v, sem).wait()
    pltpu.make_async_copy(w_hbm.at[wid], w_v, sem).wait()

    @pl.loop(0, EPW, step=C)
    def _(e):
        for k in range(C // 16):
            idx = col_v[pl.ds(e + k * 16, 16)]
            val = w_v[pl.ds(e + k * 16, 16)]
            plsc.addupdate_scatter(hist_v, [idx], val)

    pltpu.sync_copy(hist_v, stage_sh.at[sid])
    plsc.subcore_barrier()
    for t in range(NS):
        pltpu.sync_copy(stage_sh.at[t, pl.ds(sid * RPM, RPM)], mrg_v.at[t])

    @pl.loop(0, RPM, step=16)
    def _(v):
        acc = mrg_v[0, pl.ds(v, 16)]
        for t in range(1, NS):
            acc = acc + mrg_v[t, pl.ds(v, 16)]
        mrg_v[0, pl.ds(v, 16)] = acc

    pltpu.sync_copy(mrg_v.at[0], out_hbm.at[cid, pl.ds(sid * RPM, RPM)])


@functools.partial(
    pl.kernel,
    out_type=jax.ShapeDtypeStruct((NC, NPAD, FD), jnp.float32),
    mesh=_mesh,
    scratch_types=(
        [pltpu.VMEM((3, C), jnp.int32)] * 8
        + [pltpu.VMEM((C, FD), jnp.float32)] * 4
        + [pltpu.VMEM_SHARED((NPAD, FD), jnp.float32)]
        + [pltpu.SemaphoreType.DMA] * 4
        + [pltpu.SemaphoreType.DMA] * 2
        + [pltpu.SemaphoreType.DMA] * 2
    ),
    compiler_params=_sc_params,
)
def _agg_kernel(y_hbm, pk_hbm, out_hbm, pk0, pk1, pk2, pk3, pk4, pk5, pk6,
                pk7, gb0, gb1, gb2, gb3, acc_sh, gs0, gs1, gs2, gs3, ss0,
                ss1, ps0, ps1):
    cid = lax.axis_index("c")
    sid = lax.axis_index("s")
    wid = cid * NS + sid

    pkb = (pk0, pk1, pk2, pk3, pk4, pk5, pk6, pk7)
    gbb = (gb0, gb1, gb2, gb3)
    pss = (ps0, ps1)
    gss = (gs0, gs1, gs2, gs3)
    sss = (ss0, ss1)

    def issue_pk(j, jj):
        pltpu.async_copy(pk_hbm.at[wid, j], pkb[jj % 8], pss[jj % 2])

    def wait_pk(j, jj):
        pltpu.make_async_copy(pk_hbm.at[wid, j], pkb[jj % 8],
                              pss[jj % 2]).wait()

    def issue_gather(j, jj):
        pltpu.async_copy(y_hbm.at[pkb[jj % 8].at[0]], gbb[jj % 4],
                         gss[jj % 4])

    def wait_gather(j, jj):
        pltpu.make_async_copy(y_hbm.at[pkb[jj % 8].at[0]], gbb[jj % 4],
                              gss[jj % 4]).wait()

    def issue_scat(j, jj):
        pltpu.async_copy(gbb[jj % 4], acc_sh.at[pkb[jj % 8].at[1]],
                         sss[jj % 2], add=True)

    def wait_scat(j, jj):
        pltpu.make_async_copy(gbb[jj % 4], acc_sh.at[pkb[jj % 8].at[1]],
                              sss[jj % 2]).wait()

    for t in range(6):
        issue_pk(t, t)

    zero16 = jnp.zeros((16,), jnp.float32)

    @pl.loop(0, C)
    def _(r):
        for k in range(FD // 16):
            gb0[r, pl.ds(k * 16, 16)] = zero16

    for z in range(RPT // C):
        r0 = sid * RPT + z * C
        pltpu.sync_copy(gb0, acc_sh.at[pl.ds(r0, C)])

    wait_pk(0, 0)
    issue_gather(0, 0)
    wait_pk(1, 1)
    issue_gather(1, 1)
    plsc.subcore_barrier()

    def scale(jj):
        gb = gbb[jj % 4]
        wrow = pkb[jj % 8]

        @pl.loop(0, C, step=16)
        def _(g):
            wv = plsc.bitcast(wrow[2, pl.ds(g, 16)], jnp.float32)
            for l in range(16):
                w_s = wv[l]
                for k in range(FD // 16):
                    gb[g + l, pl.ds(k * 16, 16)] = (
                        gb[g + l, pl.ds(k * 16, 16)] * w_s)

    def chunk(j, jj, last):
        if jj >= 2:
            wait_scat(j - 2, jj - 2)
        wait_gather(j, jj)
        scale(jj)
        issue_scat(j, jj)
        if not last:
            @pl.when(j + 6 < NCH)
            def _():
                issue_pk(j + 6, jj + 6)

            @pl.when(j + 2 < NCH)
            def _():
                wait_pk(j + 2, jj + 2)
                issue_gather(j + 2, jj + 2)

    chunk(0, 0, False)
    chunk(1, 1, False)
    chunk(2, 2, False)

    @pl.loop(3, 115, step=8)
    def _(j):
        for t in range(8):
            chunk(j + t, 3 + t, False)

    for t in range(115, NCH - 1):
        chunk(t, t, False)
    chunk(NCH - 1, NCH - 1, True)
    wait_scat(NCH - 2, NCH - 2)
    wait_scat(NCH - 1, NCH - 1)

    plsc.subcore_barrier()
    for z in range(RPT // ZROWS):
        r0 = sid * RPT + z * ZROWS
        pltpu.sync_copy(acc_sh.at[pl.ds(r0, ZROWS)],
                        out_hbm.at[cid, pl.ds(r0, ZROWS)])


_RB = 2000
_GRID = (N // _RB,)


def _rows2(i):
    return (i, 0)


def _const2(i):
    return (0, 0)


_blk_feat = pl.BlockSpec((_RB, FD), _rows2)
_blk_w = pl.BlockSpec((FD, FD), _const2)
_blk_dis = pl.BlockSpec((_RB, 1), _rows2)
_blk_b = pl.BlockSpec((1, FD), _const2)


def _t1_body(x_ref, w_ref, d0_ref, d1_ref, y_ref, dis_ref):
    deg = d0_ref[...] + d1_ref[...]
    dis = jnp.where(deg > 0, lax.rsqrt(jnp.maximum(deg, 1e-12)), 0.0)
    dis_ref[...] = dis
    xw = jnp.dot(x_ref[...], w_ref[...], preferred_element_type=jnp.float32,
                 precision=lax.Precision.HIGHEST)
    y_ref[...] = xw * dis


_t1 = pl.pallas_call(
    _t1_body,
    grid=_GRID,
    in_specs=[_blk_feat, _blk_w, _blk_dis, _blk_dis],
    out_specs=[_blk_feat, _blk_dis],
    out_shape=[jax.ShapeDtypeStruct((N, FD), jnp.float32),
               jax.ShapeDtypeStruct((N, 1), jnp.float32)],
)


def _t2_body(p0_ref, p1_ref, dis_ref, b_ref, w_ref, y_ref):
    dis = dis_ref[...]
    h = dis * (p0_ref[...] + p1_ref[...]) + b_ref[...]
    h = jnp.maximum(h, 0.0)
    hw = jnp.dot(h, w_ref[...], preferred_element_type=jnp.float32,
                 precision=lax.Precision.HIGHEST)
    y_ref[...] = hw * dis


_t2 = pl.pallas_call(
    _t2_body,
    grid=_GRID,
    in_specs=[_blk_feat, _blk_feat, _blk_dis, _blk_b, _blk_w],
    out_specs=_blk_feat,
    out_shape=jax.ShapeDtypeStruct((N, FD), jnp.float32),
)


def _t4_body(p0_ref, p1_ref, dis_ref, b_ref, o_ref):
    o_ref[...] = dis_ref[...] * (p0_ref[...] + p1_ref[...]) + b_ref[...]


_t4 = pl.pallas_call(
    _t4_body,
    grid=_GRID,
    in_specs=[_blk_feat, _blk_feat, _blk_dis, _blk_b],
    out_specs=_blk_feat,
    out_shape=jax.ShapeDtypeStruct((N, FD), jnp.float32),
)


def kernel(x, edge_index, edge_weight, W1, b1, W2, b2, W3, b3):
    col2 = edge_index[1].reshape(NW, EPW)
    w2 = edge_weight.reshape(NW, EPW)
    pk = jnp.concatenate(
        [edge_index[0].reshape(NW, NCH, 1, C),
         edge_index[1].reshape(NW, NCH, 1, C),
         lax.bitcast_convert_type(edge_weight, jnp.int32).reshape(
             NW, NCH, 1, C)],
        axis=2)

    degp = _deg_kernel(col2, w2)
    d0 = degp[0, :N].reshape(N, 1)
    d1 = degp[1, :N].reshape(N, 1)

    y1, dis = _t1(x, W1, d0, d1)
    p1 = _agg_kernel(y1, pk)
    y2 = _t2(p1[0, :N], p1[1, :N], dis, b1.reshape(1, FD), W2)
    p2 = _agg_kernel(y2, pk)
    y3 = _t2(p2[0, :N], p2[1, :N], dis, b2.reshape(1, FD), W3)
    p3 = _agg_kernel(y3, pk)
    return _t4(p3[0, :N], p3[1, :N], dis, b3.reshape(1, FD))
